# baseline (speedup 1.0000x reference)
import functools

import jax
import jax.numpy as jnp
from jax import lax
from jax.experimental import pallas as pl
from jax.experimental.pallas import tpu as pltpu
from jax.experimental.pallas import tpu_sc as plsc

N = 10000
E = 320000
D = 128

NC = 2
NS = 16
NW = NC * NS
EPW = E // NW
CH = 80
NCHUNK = EPW // CH
BLKR = 25
NBLK = NCHUNK // BLKR
RPT = 624
TAILR = N - NS * RPT

NP = N + 8
P_EPW = 10240
P_CH = 128
P_NCHUNK = P_EPW // P_CH
P_BLKR = 16
P_NBLK = P_NCHUNK // P_BLKR

_mesh = plsc.VectorSubcoreMesh(
    core_axis_name="c", subcore_axis_name="s", num_cores=NC, num_subcores=NS
)

_f32 = jnp.float32

_GATHER_DNUMS = lax.GatherDimensionNumbers(
    offset_dims=(), collapsed_slice_dims=(0,), start_index_map=(0,))


def _lane_perm(v, idx):
    return lax.gather(v, idx[:, None], _GATHER_DNUMS, slice_sizes=(1,),
                      mode=lax.GatherScatterMode.PROMISE_IN_BOUNDS)


_BITREV4 = tuple(int(format(l, "04b")[::-1], 2) for l in range(16))


def _merge(a, b, g):
    lanes = lax.iota(jnp.int32, 16)
    mask = (lanes % (2 * g)) < g
    return (jnp.where(mask, a, _lane_perm(b, (lanes - g) % 16))
            + jnp.where(mask, _lane_perm(a, (lanes + g) % 16), b))


def _make_segsum(with_deg):
    out_type = jax.ShapeDtypeStruct((NC, N, D), _f32)
    scratch = [
        pltpu.VMEM((BLKR, CH), jnp.int32),
        pltpu.VMEM((BLKR, CH), jnp.int32),
        pltpu.VMEM((CH, D), _f32),
        pltpu.VMEM((CH, D), _f32),
        pltpu.SemaphoreType.DMA,
        pltpu.SemaphoreType.DMA,
        pltpu.VMEM_SHARED((N, D), _f32),
    ]
    if with_deg:
        out_type = (out_type, jax.ShapeDtypeStruct((NC * N,), _f32))
        scratch += [
            pltpu.VMEM((CH,), _f32),
            pltpu.VMEM_SHARED((N,), _f32),
            pltpu.VMEM((RPT + TAILR,), _f32),
        ]

    @functools.partial(
        pl.kernel, out_type=out_type, mesh=_mesh,
        scratch_types=tuple(scratch),
    )
    def segsum(*refs):
        if with_deg:
            (feat, src4d, dst4d, znd, acc_out, deg_out,
             idx_s, idx_d, rows0, rows1, sem0, sem1, acc_sh,
             ones_v, deg_sh, deg_buf) = refs
        else:
            (feat, src4d, dst4d, znd, acc_out,
             idx_s, idx_d, rows0, rows1, sem0, sem1, acc_sh) = refs
        rows = (rows0, rows1)
        sems = (sem0, sem1)

        c = lax.axis_index("c")
        s = lax.axis_index("s")
        w = c * NS + s

        pltpu.sync_copy(znd.at[pl.ds(s * RPT, RPT)],
                        acc_sh.at[pl.ds(s * RPT, RPT)])

        @pl.when(s == NS - 1)
        def _():
            pltpu.sync_copy(znd.at[pl.ds(NS * RPT, TAILR)],
                            acc_sh.at[pl.ds(NS * RPT, TAILR)])

        if with_deg:
            for j in range((RPT + TAILR) // 16):
                deg_buf[pl.ds(j * 16, 16)] = jnp.zeros((16,), _f32)
            pltpu.sync_copy(deg_buf.at[pl.ds(0, RPT)],
                            deg_sh.at[pl.ds(s * RPT, RPT)])

            @pl.when(s == NS - 1)
            def _():
                pltpu.sync_copy(deg_buf.at[pl.ds(0, TAILR)],
                                deg_sh.at[pl.ds(NS * RPT, TAILR)])

            for j in range(CH // 16):
                ones_v[pl.ds(j * 16, 16)] = jnp.ones((16,), _f32)
        plsc.subcore_barrier()

        def block(blk, carry0):
            pltpu.sync_copy(src4d.at[w, blk], idx_s)
            pltpu.sync_copy(dst4d.at[w, blk], idx_d)

            for b in range(2):
                pltpu.async_copy(feat.at[idx_s.at[b]], rows[b], sems[b])

            def consume(cc, b):
                pltpu.make_async_copy(feat.at[idx_s.at[cc]], rows[b],
                                      sems[b]).wait()
                pltpu.sync_copy(rows[b], acc_sh.at[idx_d.at[cc]], add=True)
                if with_deg:
                    pltpu.sync_copy(ones_v, deg_sh.at[idx_d.at[cc]],
                                    add=True)

                @pl.when(cc + 2 < BLKR)
                def _():
                    pltpu.async_copy(feat.at[idx_s.at[cc + 2]], rows[b],
                                     sems[b])

            def pairj(j, carry):
                consume(j * 2, 0)

                @pl.when(j * 2 + 1 < BLKR)
                def _():
                    consume(j * 2 + 1, 1)

                return carry

            lax.fori_loop(0, (BLKR + 1) // 2, pairj, 0)
            return carry0

        lax.fori_loop(0, NBLK, block, 0)
        plsc.subcore_barrier()

        pltpu.sync_copy(acc_sh.at[pl.ds(s * RPT, RPT)],
                        acc_out.at[c, pl.ds(s * RPT, RPT)])

        @pl.when(s == NS - 1)
        def _():
            pltpu.sync_copy(acc_sh.at[pl.ds(NS * RPT, TAILR)],
                            acc_out.at[c, pl.ds(NS * RPT, TAILR)])

        if with_deg:
            pltpu.sync_copy(deg_sh.at[pl.ds(s * RPT, RPT)],
                            deg_buf.at[pl.ds(0, RPT)])
            pltpu.sync_copy(deg_buf.at[pl.ds(0, RPT)],
                            deg_out.at[pl.ds(c * N + s * RPT, RPT)])

            @pl.when(s == NS - 1)
            def _():
                pltpu.sync_copy(deg_sh.at[pl.ds(NS * RPT, TAILR)],
                                deg_buf.at[pl.ds(0, TAILR)])
                pltpu.sync_copy(deg_buf.at[pl.ds(0, TAILR)],
                                deg_out.at[pl.ds(c * N + NS * RPT, TAILR)])

    return segsum


_segsum_deg = _make_segsum(True)
_segsum = _make_segsum(False)


@functools.partial(
    pl.kernel,
    out_type=jax.ShapeDtypeStruct((NW * P_EPW,), _f32),
    mesh=_mesh,
    scratch_types=(
        pltpu.VMEM((P_BLKR, P_CH), jnp.int32),
        pltpu.VMEM((P_BLKR, P_CH), jnp.int32),
        pltpu.VMEM((P_CH, D), _f32),
        pltpu.VMEM((P_CH, D), _f32),
        pltpu.VMEM((P_CH, D), _f32),
        pltpu.VMEM((P_CH, D), _f32),
        pltpu.SemaphoreType.DMA,
        pltpu.SemaphoreType.DMA,
        pltpu.SemaphoreType.DMA,
        pltpu.SemaphoreType.DMA,
        pltpu.VMEM((P_CH,), _f32),
        pltpu.VMEM((P_CH,), _f32),
        pltpu.SemaphoreType.DMA,
        pltpu.SemaphoreType.DMA,
    ),
)
def _edge_dot(feat, src4d, dst4d, out, idx_s, idx_d, rs0, rs1, rd0, rd1,
              ss0, ss1, sd0, sd1, ov0, ov1, so0, so1):
    rows_s = (rs0, rs1)
    rows_d = (rd0, rd1)
    sem_s = (ss0, ss1)
    sem_d = (sd0, sd1)
    out_v = (ov0, ov1)
    sem_o = (so0, so1)
    w = lax.axis_index("c") * NS + lax.axis_index("s")

    def block(blk, carry0):
        pltpu.sync_copy(src4d.at[w, blk], idx_s)
        pltpu.sync_copy(dst4d.at[w, blk], idx_d)

        for b in range(2):
            pltpu.async_copy(feat.at[idx_s.at[b]], rows_s[b], sem_s[b])
            pltpu.async_copy(feat.at[idx_d.at[b]], rows_d[b], sem_d[b])

        def consume(cc, b):
            g = blk * P_BLKR + cc

            @pl.when(g >= 2)
            def _():
                pltpu.make_async_copy(out_v[b], out.at[pl.ds(0, P_CH)],
                                      sem_o[b]).wait()

            pltpu.make_async_copy(feat.at[idx_s.at[cc]], rows_s[b],
                                  sem_s[b]).wait()
            pltpu.make_async_copy(feat.at[idx_d.at[cc]], rows_d[b],
                                  sem_d[b]).wait()

            def group16(q, carry2):
                vs = [None] * 16
                for m in range(16):
                    e = q * 16 + m
                    v = (rows_s[b][e, pl.ds(0, 16)]
                         * rows_d[b][e, pl.ds(0, 16)])
                    for j in range(1, D // 16):
                        v = v + (rows_s[b][e, pl.ds(j * 16, 16)]
                                 * rows_d[b][e, pl.ds(j * 16, 16)])
                    vs[_BITREV4[m]] = v
                for g in (8, 4, 2, 1):
                    vs = [_merge(vs[2 * i], vs[2 * i + 1], g)
                          for i in range(len(vs) // 2)]
                out_v[b][pl.ds(q * 16, 16)] = vs[0]
                return carry2

            lax.fori_loop(0, P_CH // 16, group16, 0)
            pltpu.async_copy(out_v[b],
                             out.at[pl.ds((w * P_NCHUNK + g) * P_CH, P_CH)],
                             sem_o[b])

            @pl.when(cc + 2 < P_BLKR)
            def _():
                pltpu.async_copy(feat.at[idx_s.at[cc + 2]], rows_s[b],
                                 sem_s[b])
                pltpu.async_copy(feat.at[idx_d.at[cc + 2]], rows_d[b],
                                 sem_d[b])

        def pairj(j, carry):
            consume(j * 2, 0)
            consume(j * 2 + 1, 1)
            return carry

        lax.fori_loop(0, P_BLKR // 2, pairj, 0)
        return carry0

    lax.fori_loop(0, P_NBLK, block, 0)
    for b in range(2):
        pltpu.make_async_copy(out_v[b], out.at[pl.ds(0, P_CH)],
                              sem_o[b]).wait()


RB = 1000


def _tc_layer_body(relu, x_ref, a0, a1, d0, d1, ws, wn, b, o_ref):
    deg = jnp.maximum(d0[...] + d1[...], 1.0)
    neigh = (a0[...] + a1[...]) / deg
    h = (jnp.dot(x_ref[...], ws[...], preferred_element_type=_f32)
         + jnp.dot(neigh, wn[...], preferred_element_type=_f32)
         + b[...])
    if relu:
        h = jnp.maximum(h, 0.0)
    o_ref[...] = h


def _tc_layer(x, acc, deg, w_self, w_neigh, b, relu):
    row_spec = pl.BlockSpec((RB, D), lambda i: (i, 0))
    col_spec = pl.BlockSpec((RB, 1), lambda i: (i, 0))
    w_spec = pl.BlockSpec((D, D), lambda i: (0, 0))
    b_spec = pl.BlockSpec((1, D), lambda i: (0, 0))
    return pl.pallas_call(
        functools.partial(_tc_layer_body, relu),
        grid=(N // RB,),
        in_specs=[row_spec, row_spec, row_spec, col_spec, col_spec,
                  w_spec, w_spec, b_spec],
        out_specs=row_spec,
        out_shape=jax.ShapeDtypeStruct((N, D), _f32),
    )(x, acc[0], acc[1], deg[0].reshape(N, 1), deg[1].reshape(N, 1),
      w_self, w_neigh, b.reshape(1, D))


def kernel(x, edge_index, W_self1, W_neigh1, b1, W_self2, W_neigh2, b2):
    src4d = edge_index[0].reshape(NW, NBLK, BLKR, CH)
    dst4d = edge_index[1].reshape(NW, NBLK, BLKR, CH)
    znd = jnp.zeros((N, D), _f32)

    npad = P_EPW - EPW
    pad_idx = N + (jnp.arange(npad, dtype=jnp.int32) % (NP - N))
    psrc = jnp.concatenate(
        [edge_index[0].reshape(NW, EPW), jnp.broadcast_to(pad_idx, (NW, npad))],
        axis=1).reshape(NW, P_NBLK, P_BLKR, P_CH)
    pdst = jnp.concatenate(
        [edge_index[1].reshape(NW, EPW), jnp.broadcast_to(pad_idx, (NW, npad))],
        axis=1).reshape(NW, P_NBLK, P_BLKR, P_CH)

    accx, degp = _segsum_deg(x, src4d, dst4d, znd)
    degp = degp.reshape(NC, N)
    h = _tc_layer(x, accx, degp, W_self1, W_neigh1, b1, relu=True)
    acch = _segsum(h, src4d, dst4d, znd)
    h2 = _tc_layer(h, acch, degp, W_self2, W_neigh2, b2, relu=False)
    score = _edge_dot(jnp.pad(h2, ((0, NP - N), (0, 0))), psrc, pdst)
    return score.reshape(NW, P_EPW)[:, :EPW].reshape(E, 1)


# speedup vs baseline: 1.1316x; 1.1316x over previous
import functools

import jax
import jax.numpy as jnp
from jax import lax
from jax.experimental import pallas as pl
from jax.experimental.pallas import tpu as pltpu
from jax.experimental.pallas import tpu_sc as plsc

N = 10000
E = 320000
D = 128

NC = 2
NS = 16
NW = NC * NS
EPW = E // NW
CH = 80
NCHUNK = EPW // CH
BLKR = 25
NBLK = NCHUNK // BLKR
RPT = 624
TAILR = N - NS * RPT

_mesh = plsc.VectorSubcoreMesh(
    core_axis_name="c", subcore_axis_name="s", num_cores=NC, num_subcores=NS
)

_f32 = jnp.float32

_GATHER_DNUMS = lax.GatherDimensionNumbers(
    offset_dims=(), collapsed_slice_dims=(0,), start_index_map=(0,))


def _lane_perm(v, idx):
    return lax.gather(v, idx[:, None], _GATHER_DNUMS, slice_sizes=(1,),
                      mode=lax.GatherScatterMode.PROMISE_IN_BOUNDS)


_BITREV4 = tuple(int(format(l, "04b")[::-1], 2) for l in range(16))


def _merge(a, b, g):
    lanes = lax.iota(jnp.int32, 16)
    mask = (lanes % (2 * g)) < g
    return (jnp.where(mask, a, _lane_perm(b, (lanes - g) % 16))
            + jnp.where(mask, _lane_perm(a, (lanes + g) % 16), b))


def _make_segsum(with_deg):
    out_type = jax.ShapeDtypeStruct((NC, N, D), _f32)
    scratch = [
        pltpu.VMEM((BLKR, CH), jnp.int32),
        pltpu.VMEM((BLKR, CH), jnp.int32),
        pltpu.VMEM((CH, D), _f32),
        pltpu.VMEM((CH, D), _f32),
        pltpu.SemaphoreType.DMA,
        pltpu.SemaphoreType.DMA,
        pltpu.VMEM_SHARED((N, D), _f32),
    ]
    if with_deg:
        out_type = (out_type, jax.ShapeDtypeStruct((NC * N,), _f32))
        scratch += [
            pltpu.VMEM((CH,), _f32),
            pltpu.VMEM_SHARED((N,), _f32),
            pltpu.VMEM((RPT + TAILR,), _f32),
        ]

    @functools.partial(
        pl.kernel, out_type=out_type, mesh=_mesh,
        scratch_types=tuple(scratch),
    )
    def segsum(*refs):
        if with_deg:
            (feat, src4d, dst4d, znd, acc_out, deg_out,
             idx_s, idx_d, rows0, rows1, sem0, sem1, acc_sh,
             ones_v, deg_sh, deg_buf) = refs
        else:
            (feat, src4d, dst4d, znd, acc_out,
             idx_s, idx_d, rows0, rows1, sem0, sem1, acc_sh) = refs
        rows = (rows0, rows1)
        sems = (sem0, sem1)

        c = lax.axis_index("c")
        s = lax.axis_index("s")
        w = c * NS + s

        pltpu.sync_copy(znd.at[pl.ds(s * RPT, RPT)],
                        acc_sh.at[pl.ds(s * RPT, RPT)])

        @pl.when(s == NS - 1)
        def _():
            pltpu.sync_copy(znd.at[pl.ds(NS * RPT, TAILR)],
                            acc_sh.at[pl.ds(NS * RPT, TAILR)])

        if with_deg:
            for j in range((RPT + TAILR) // 16):
                deg_buf[pl.ds(j * 16, 16)] = jnp.zeros((16,), _f32)
            pltpu.sync_copy(deg_buf.at[pl.ds(0, RPT)],
                            deg_sh.at[pl.ds(s * RPT, RPT)])

            @pl.when(s == NS - 1)
            def _():
                pltpu.sync_copy(deg_buf.at[pl.ds(0, TAILR)],
                                deg_sh.at[pl.ds(NS * RPT, TAILR)])

            for j in range(CH // 16):
                ones_v[pl.ds(j * 16, 16)] = jnp.ones((16,), _f32)
        plsc.subcore_barrier()

        def block(blk, carry0):
            pltpu.sync_copy(src4d.at[w, blk], idx_s)
            pltpu.sync_copy(dst4d.at[w, blk], idx_d)

            for b in range(2):
                pltpu.async_copy(feat.at[idx_s.at[b]], rows[b], sems[b])

            def consume(cc, b):
                pltpu.make_async_copy(feat.at[idx_s.at[cc]], rows[b],
                                      sems[b]).wait()
                pltpu.sync_copy(rows[b], acc_sh.at[idx_d.at[cc]], add=True)
                if with_deg:
                    pltpu.sync_copy(ones_v, deg_sh.at[idx_d.at[cc]],
                                    add=True)

                @pl.when(cc + 2 < BLKR)
                def _():
                    pltpu.async_copy(feat.at[idx_s.at[cc + 2]], rows[b],
                                     sems[b])

            def pairj(j, carry):
                consume(j * 2, 0)

                @pl.when(j * 2 + 1 < BLKR)
                def _():
                    consume(j * 2 + 1, 1)

                return carry

            lax.fori_loop(0, (BLKR + 1) // 2, pairj, 0)
            return carry0

        lax.fori_loop(0, NBLK, block, 0)
        plsc.subcore_barrier()

        pltpu.sync_copy(acc_sh.at[pl.ds(s * RPT, RPT)],
                        acc_out.at[c, pl.ds(s * RPT, RPT)])

        @pl.when(s == NS - 1)
        def _():
            pltpu.sync_copy(acc_sh.at[pl.ds(NS * RPT, TAILR)],
                            acc_out.at[c, pl.ds(NS * RPT, TAILR)])

        if with_deg:
            pltpu.sync_copy(deg_sh.at[pl.ds(s * RPT, RPT)],
                            deg_buf.at[pl.ds(0, RPT)])
            pltpu.sync_copy(deg_buf.at[pl.ds(0, RPT)],
                            deg_out.at[pl.ds(c * N + s * RPT, RPT)])

            @pl.when(s == NS - 1)
            def _():
                pltpu.sync_copy(deg_sh.at[pl.ds(NS * RPT, TAILR)],
                                deg_buf.at[pl.ds(0, TAILR)])
                pltpu.sync_copy(deg_buf.at[pl.ds(0, TAILR)],
                                deg_out.at[pl.ds(c * N + NS * RPT, TAILR)])

    return segsum


_segsum_deg = _make_segsum(True)
_segsum = _make_segsum(False)


@functools.partial(
    pl.kernel,
    out_type=jax.ShapeDtypeStruct((E,), _f32),
    mesh=_mesh,
    scratch_types=(
        pltpu.VMEM((BLKR, CH), jnp.int32),
        pltpu.VMEM((BLKR, CH), jnp.int32),
        pltpu.VMEM((CH, D), _f32),
        pltpu.VMEM((CH, D), _f32),
        pltpu.VMEM((CH, D), _f32),
        pltpu.VMEM((CH, D), _f32),
        pltpu.SemaphoreType.DMA,
        pltpu.SemaphoreType.DMA,
        pltpu.SemaphoreType.DMA,
        pltpu.SemaphoreType.DMA,
        pltpu.VMEM((CH,), _f32),
        pltpu.VMEM((CH,), _f32),
        pltpu.SemaphoreType.DMA,
        pltpu.SemaphoreType.DMA,
    ),
)
def _edge_dot(feat, src4d, dst4d, out, idx_s, idx_d, rs0, rs1, rd0, rd1,
              ss0, ss1, sd0, sd1, ov0, ov1, so0, so1):
    rows_s = (rs0, rs1)
    rows_d = (rd0, rd1)
    sem_s = (ss0, ss1)
    sem_d = (sd0, sd1)
    out_v = (ov0, ov1)
    sem_o = (so0, so1)
    w = lax.axis_index("c") * NS + lax.axis_index("s")

    def block(blk, carry0):
        pltpu.sync_copy(src4d.at[w, blk], idx_s)
        pltpu.sync_copy(dst4d.at[w, blk], idx_d)

        for b in range(2):
            pltpu.async_copy(feat.at[idx_s.at[b]], rows_s[b], sem_s[b])
            pltpu.async_copy(feat.at[idx_d.at[b]], rows_d[b], sem_d[b])

        def consume(cc, b):
            g = blk * BLKR + cc

            @pl.when(g >= 2)
            def _():
                pltpu.make_async_copy(out_v[b], out.at[pl.ds(0, CH)],
                                      sem_o[b]).wait()

            pltpu.make_async_copy(feat.at[idx_s.at[cc]], rows_s[b],
                                  sem_s[b]).wait()
            pltpu.make_async_copy(feat.at[idx_d.at[cc]], rows_d[b],
                                  sem_d[b]).wait()

            def group16(q, carry2):
                vs = [None] * 16
                for m in range(16):
                    e = q * 16 + m
                    v = (rows_s[b][e, pl.ds(0, 16)]
                         * rows_d[b][e, pl.ds(0, 16)])
                    for j in range(1, D // 16):
                        v = v + (rows_s[b][e, pl.ds(j * 16, 16)]
                                 * rows_d[b][e, pl.ds(j * 16, 16)])
                    vs[_BITREV4[m]] = v
                for g in (8, 4, 2, 1):
                    vs = [_merge(vs[2 * i], vs[2 * i + 1], g)
                          for i in range(len(vs) // 2)]
                out_v[b][pl.ds(q * 16, 16)] = vs[0]
                return carry2

            lax.fori_loop(0, CH // 16, group16, 0)
            pltpu.async_copy(out_v[b],
                             out.at[pl.ds((w * NCHUNK + g) * CH, CH)],
                             sem_o[b])

            @pl.when(cc + 2 < BLKR)
            def _():
                pltpu.async_copy(feat.at[idx_s.at[cc + 2]], rows_s[b],
                                 sem_s[b])
                pltpu.async_copy(feat.at[idx_d.at[cc + 2]], rows_d[b],
                                 sem_d[b])

        def pairj(j, carry):
            consume(j * 2, 0)

            @pl.when(j * 2 + 1 < BLKR)
            def _():
                consume(j * 2 + 1, 1)

            return carry

        lax.fori_loop(0, (BLKR + 1) // 2, pairj, 0)
        return carry0

    lax.fori_loop(0, NBLK, block, 0)
    for b in range(2):
        pltpu.make_async_copy(out_v[b], out.at[pl.ds(0, CH)],
                              sem_o[b]).wait()


RB = 1000


def _tc_layer_body(relu, x_ref, a0, a1, d0, d1, ws, wn, b, o_ref):
    deg = jnp.maximum(d0[...] + d1[...], 1.0)
    neigh = (a0[...] + a1[...]) / deg
    h = (jnp.dot(x_ref[...], ws[...], preferred_element_type=_f32)
         + jnp.dot(neigh, wn[...], preferred_element_type=_f32)
         + b[...])
    if relu:
        h = jnp.maximum(h, 0.0)
    o_ref[...] = h


def _tc_layer(x, acc, deg, w_self, w_neigh, b, relu):
    row_spec = pl.BlockSpec((RB, D), lambda i: (i, 0))
    col_spec = pl.BlockSpec((RB, 1), lambda i: (i, 0))
    w_spec = pl.BlockSpec((D, D), lambda i: (0, 0))
    b_spec = pl.BlockSpec((1, D), lambda i: (0, 0))
    return pl.pallas_call(
        functools.partial(_tc_layer_body, relu),
        grid=(N // RB,),
        in_specs=[row_spec, row_spec, row_spec, col_spec, col_spec,
                  w_spec, w_spec, b_spec],
        out_specs=row_spec,
        out_shape=jax.ShapeDtypeStruct((N, D), _f32),
    )(x, acc[0], acc[1], deg[0].reshape(N, 1), deg[1].reshape(N, 1),
      w_self, w_neigh, b.reshape(1, D))


def kernel(x, edge_index, W_self1, W_neigh1, b1, W_self2, W_neigh2, b2):
    src4d = edge_index[0].reshape(NW, NBLK, BLKR, CH)
    dst4d = edge_index[1].reshape(NW, NBLK, BLKR, CH)
    znd = jnp.zeros((N, D), _f32)

    accx, degp = _segsum_deg(x, src4d, dst4d, znd)
    degp = degp.reshape(NC, N)
    h = _tc_layer(x, accx, degp, W_self1, W_neigh1, b1, relu=True)
    acch = _segsum(h, src4d, dst4d, znd)
    h2 = _tc_layer(h, acch, degp, W_self2, W_neigh2, b2, relu=False)
    score = _edge_dot(h2, src4d, dst4d)
    return score.reshape(E, 1)


# speedup vs baseline: 1.2043x; 1.0643x over previous
import functools

import jax
import jax.numpy as jnp
from jax import lax
from jax.experimental import pallas as pl
from jax.experimental.pallas import tpu as pltpu
from jax.experimental.pallas import tpu_sc as plsc

N = 10000
E = 320000
D = 128

NC = 2
NS = 16
NW = NC * NS
EPW = E // NW
CH = 80
NCHUNK = EPW // CH
BLKR = 25
NBLK = NCHUNK // BLKR
RPT = 624
TAILR = N - NS * RPT

_mesh = plsc.VectorSubcoreMesh(
    core_axis_name="c", subcore_axis_name="s", num_cores=NC, num_subcores=NS
)

_f32 = jnp.float32

_GATHER_DNUMS = lax.GatherDimensionNumbers(
    offset_dims=(), collapsed_slice_dims=(0,), start_index_map=(0,))


def _lane_perm(v, idx):
    return lax.gather(v, idx[:, None], _GATHER_DNUMS, slice_sizes=(1,),
                      mode=lax.GatherScatterMode.PROMISE_IN_BOUNDS)


_BITREV4 = tuple(int(format(l, "04b")[::-1], 2) for l in range(16))


def _merge(a, b, g):
    lanes = lax.iota(jnp.int32, 16)
    mask = (lanes % (2 * g)) < g
    return (jnp.where(mask, a, _lane_perm(b, (lanes - g) % 16))
            + jnp.where(mask, _lane_perm(a, (lanes + g) % 16), b))


def _make_segsum(with_deg):
    out_type = jax.ShapeDtypeStruct((NC, N, D), _f32)
    scratch = [
        pltpu.VMEM((BLKR, CH), jnp.int32),
        pltpu.VMEM((BLKR, CH), jnp.int32),
        pltpu.VMEM((CH, D), _f32),
        pltpu.VMEM((CH, D), _f32),
        pltpu.VMEM((CH, D), _f32),
        pltpu.SemaphoreType.DMA,
        pltpu.SemaphoreType.DMA,
        pltpu.SemaphoreType.DMA,
        pltpu.SemaphoreType.DMA,
        pltpu.SemaphoreType.DMA,
        pltpu.SemaphoreType.DMA,
        pltpu.VMEM_SHARED((N, D), _f32),
    ]
    if with_deg:
        out_type = (out_type, jax.ShapeDtypeStruct((NC * N,), _f32))
        scratch += [
            pltpu.VMEM((CH,), _f32),
            pltpu.VMEM_SHARED((N,), _f32),
            pltpu.VMEM((RPT + TAILR,), _f32),
        ]

    @functools.partial(
        pl.kernel, out_type=out_type, mesh=_mesh,
        scratch_types=tuple(scratch),
    )
    def segsum(*refs):
        if with_deg:
            (feat, src4d, dst4d, znd, acc_out, deg_out,
             idx_s, idx_d, rows0, rows1, rows2, sg0, sg1, sg2,
             sc0, sc1, sc2, acc_sh, ones_v, deg_sh, deg_buf) = refs
        else:
            (feat, src4d, dst4d, znd, acc_out,
             idx_s, idx_d, rows0, rows1, rows2, sg0, sg1, sg2,
             sc0, sc1, sc2, acc_sh) = refs
        rows = (rows0, rows1, rows2)
        sems = (sg0, sg1, sg2)
        sem_sc = (sc0, sc1, sc2)

        c = lax.axis_index("c")
        s = lax.axis_index("s")
        w = c * NS + s

        pltpu.sync_copy(znd.at[pl.ds(s * RPT, RPT)],
                        acc_sh.at[pl.ds(s * RPT, RPT)])

        @pl.when(s == NS - 1)
        def _():
            pltpu.sync_copy(znd.at[pl.ds(NS * RPT, TAILR)],
                            acc_sh.at[pl.ds(NS * RPT, TAILR)])

        if with_deg:
            for j in range((RPT + TAILR) // 16):
                deg_buf[pl.ds(j * 16, 16)] = jnp.zeros((16,), _f32)
            pltpu.sync_copy(deg_buf.at[pl.ds(0, RPT)],
                            deg_sh.at[pl.ds(s * RPT, RPT)])

            @pl.when(s == NS - 1)
            def _():
                pltpu.sync_copy(deg_buf.at[pl.ds(0, TAILR)],
                                deg_sh.at[pl.ds(NS * RPT, TAILR)])

            for j in range(CH // 16):
                ones_v[pl.ds(j * 16, 16)] = jnp.ones((16,), _f32)
        plsc.subcore_barrier()

        def block(blk, carry0):
            pltpu.sync_copy(src4d.at[w, blk], idx_s)
            pltpu.sync_copy(dst4d.at[w, blk], idx_d)

            for b in range(2):
                pltpu.async_copy(feat.at[idx_s.at[b]], rows[b], sems[b])

            def consume(cc, b):
                pltpu.make_async_copy(feat.at[idx_s.at[cc]], rows[b],
                                      sems[b]).wait()
                pltpu.async_copy(rows[b], acc_sh.at[idx_d.at[cc]],
                                 sem_sc[b], add=True)
                if with_deg:
                    pltpu.sync_copy(ones_v, deg_sh.at[idx_d.at[cc]],
                                    add=True)

                b2 = (b + 2) % 3

                @pl.when(cc + 2 < BLKR)
                def _():
                    @pl.when(cc >= 1)
                    def _():
                        pltpu.make_async_copy(
                            rows[b2], acc_sh.at[idx_d.at[cc - 1]],
                            sem_sc[b2]).wait()

                    pltpu.async_copy(feat.at[idx_s.at[cc + 2]], rows[b2],
                                     sems[b2])

            def triple(j, carry):
                for t in range(3):
                    consume(j * 3 + t, t)
                return carry

            lax.fori_loop(0, BLKR // 3, triple, 0)
            consume(BLKR - 1, (BLKR - 1) % 3)
            for cc in (BLKR - 3, BLKR - 2, BLKR - 1):
                b = cc % 3
                pltpu.make_async_copy(rows[b], acc_sh.at[idx_d.at[cc]],
                                      sem_sc[b]).wait()
            return carry0

        lax.fori_loop(0, NBLK, block, 0)
        plsc.subcore_barrier()

        pltpu.sync_copy(acc_sh.at[pl.ds(s * RPT, RPT)],
                        acc_out.at[c, pl.ds(s * RPT, RPT)])

        @pl.when(s == NS - 1)
        def _():
            pltpu.sync_copy(acc_sh.at[pl.ds(NS * RPT, TAILR)],
                            acc_out.at[c, pl.ds(NS * RPT, TAILR)])

        if with_deg:
            pltpu.sync_copy(deg_sh.at[pl.ds(s * RPT, RPT)],
                            deg_buf.at[pl.ds(0, RPT)])
            pltpu.sync_copy(deg_buf.at[pl.ds(0, RPT)],
                            deg_out.at[pl.ds(c * N + s * RPT, RPT)])

            @pl.when(s == NS - 1)
            def _():
                pltpu.sync_copy(deg_sh.at[pl.ds(NS * RPT, TAILR)],
                                deg_buf.at[pl.ds(0, TAILR)])
                pltpu.sync_copy(deg_buf.at[pl.ds(0, TAILR)],
                                deg_out.at[pl.ds(c * N + NS * RPT, TAILR)])

    return segsum


_segsum_deg = _make_segsum(True)
_segsum = _make_segsum(False)


@functools.partial(
    pl.kernel,
    out_type=jax.ShapeDtypeStruct((E,), _f32),
    mesh=_mesh,
    scratch_types=(
        pltpu.VMEM((BLKR, CH), jnp.int32),
        pltpu.VMEM((BLKR, CH), jnp.int32),
        pltpu.VMEM((CH, D), _f32),
        pltpu.VMEM((CH, D), _f32),
        pltpu.VMEM((CH, D), _f32),
        pltpu.VMEM((CH, D), _f32),
        pltpu.SemaphoreType.DMA,
        pltpu.SemaphoreType.DMA,
        pltpu.SemaphoreType.DMA,
        pltpu.SemaphoreType.DMA,
        pltpu.VMEM((CH,), _f32),
        pltpu.VMEM((CH,), _f32),
        pltpu.SemaphoreType.DMA,
        pltpu.SemaphoreType.DMA,
    ),
)
def _edge_dot(feat, src4d, dst4d, out, idx_s, idx_d, rs0, rs1, rd0, rd1,
              ss0, ss1, sd0, sd1, ov0, ov1, so0, so1):
    rows_s = (rs0, rs1)
    rows_d = (rd0, rd1)
    sem_s = (ss0, ss1)
    sem_d = (sd0, sd1)
    out_v = (ov0, ov1)
    sem_o = (so0, so1)
    w = lax.axis_index("c") * NS + lax.axis_index("s")

    def block(blk, carry0):
        pltpu.sync_copy(src4d.at[w, blk], idx_s)
        pltpu.sync_copy(dst4d.at[w, blk], idx_d)

        for b in range(2):
            pltpu.async_copy(feat.at[idx_s.at[b]], rows_s[b], sem_s[b])
            pltpu.async_copy(feat.at[idx_d.at[b]], rows_d[b], sem_d[b])

        def consume(cc, b):
            g = blk * BLKR + cc

            @pl.when(g >= 2)
            def _():
                pltpu.make_async_copy(out_v[b], out.at[pl.ds(0, CH)],
                                      sem_o[b]).wait()

            pltpu.make_async_copy(feat.at[idx_s.at[cc]], rows_s[b],
                                  sem_s[b]).wait()
            pltpu.make_async_copy(feat.at[idx_d.at[cc]], rows_d[b],
                                  sem_d[b]).wait()

            def group16(q, carry2):
                vs = [None] * 16
                for m in range(16):
                    e = q * 16 + m
                    v = (rows_s[b][e, pl.ds(0, 16)]
                         * rows_d[b][e, pl.ds(0, 16)])
                    for j in range(1, D // 16):
                        v = v + (rows_s[b][e, pl.ds(j * 16, 16)]
                                 * rows_d[b][e, pl.ds(j * 16, 16)])
                    vs[_BITREV4[m]] = v
                for g in (8, 4, 2, 1):
                    vs = [_merge(vs[2 * i], vs[2 * i + 1], g)
                          for i in range(len(vs) // 2)]
                out_v[b][pl.ds(q * 16, 16)] = vs[0]
                return carry2

            lax.fori_loop(0, CH // 16, group16, 0)
            pltpu.async_copy(out_v[b],
                             out.at[pl.ds((w * NCHUNK + g) * CH, CH)],
                             sem_o[b])

            @pl.when(cc + 2 < BLKR)
            def _():
                pltpu.async_copy(feat.at[idx_s.at[cc + 2]], rows_s[b],
                                 sem_s[b])
                pltpu.async_copy(feat.at[idx_d.at[cc + 2]], rows_d[b],
                                 sem_d[b])

        def pairj(j, carry):
            consume(j * 2, 0)

            @pl.when(j * 2 + 1 < BLKR)
            def _():
                consume(j * 2 + 1, 1)

            return carry

        lax.fori_loop(0, (BLKR + 1) // 2, pairj, 0)
        return carry0

    lax.fori_loop(0, NBLK, block, 0)
    for b in range(2):
        pltpu.make_async_copy(out_v[b], out.at[pl.ds(0, CH)],
                              sem_o[b]).wait()


RB = 1000


def _tc_layer_body(relu, x_ref, a0, a1, d0, d1, ws, wn, b, o_ref):
    deg = jnp.maximum(d0[...] + d1[...], 1.0)
    neigh = (a0[...] + a1[...]) / deg
    h = (jnp.dot(x_ref[...], ws[...], preferred_element_type=_f32)
         + jnp.dot(neigh, wn[...], preferred_element_type=_f32)
         + b[...])
    if relu:
        h = jnp.maximum(h, 0.0)
    o_ref[...] = h


def _tc_layer(x, acc, deg, w_self, w_neigh, b, relu):
    row_spec = pl.BlockSpec((RB, D), lambda i: (i, 0))
    col_spec = pl.BlockSpec((RB, 1), lambda i: (i, 0))
    w_spec = pl.BlockSpec((D, D), lambda i: (0, 0))
    b_spec = pl.BlockSpec((1, D), lambda i: (0, 0))
    return pl.pallas_call(
        functools.partial(_tc_layer_body, relu),
        grid=(N // RB,),
        in_specs=[row_spec, row_spec, row_spec, col_spec, col_spec,
                  w_spec, w_spec, b_spec],
        out_specs=row_spec,
        out_shape=jax.ShapeDtypeStruct((N, D), _f32),
    )(x, acc[0], acc[1], deg[0].reshape(N, 1), deg[1].reshape(N, 1),
      w_self, w_neigh, b.reshape(1, D))


def kernel(x, edge_index, W_self1, W_neigh1, b1, W_self2, W_neigh2, b2):
    src4d = edge_index[0].reshape(NW, NBLK, BLKR, CH)
    dst4d = edge_index[1].reshape(NW, NBLK, BLKR, CH)
    znd = jnp.zeros((N, D), _f32)

    accx, degp = _segsum_deg(x, src4d, dst4d, znd)
    degp = degp.reshape(NC, N)
    h = _tc_layer(x, accx, degp, W_self1, W_neigh1, b1, relu=True)
    acch = _segsum(h, src4d, dst4d, znd)
    h2 = _tc_layer(h, acch, degp, W_self2, W_neigh2, b2, relu=False)
    score = _edge_dot(h2, src4d, dst4d)
    return score.reshape(E, 1)
